# Initial kernel scaffold; baseline (speedup 1.0000x reference)
import jax
import jax.numpy as jnp
from jax.experimental import pallas as pl


def kernel(query_vec, qclass_vec, pvs, query_weight, label, product_idx):
    raise NotImplementedError("write your pallas kernel here")


# speedup vs baseline: 1.4156x; 1.4156x over previous
import functools

import jax
import jax.numpy as jnp
from jax import lax
from jax.experimental import pallas as pl
from jax.experimental.pallas import tpu as pltpu
from jax.experimental.pallas import tpu_sc as plsc

N_PRODUCTS = 100000
N_DIM = 1024
BATCH = 16384
N_LABELS = 3

NC, NS, L = 2, 16, 16
NW = NC * NS
RPW = BATCH // NW
K = 8
NBLK = RPW // K
NCH = N_DIM // L


def _vrsqrt(x):
    i = lax.bitcast_convert_type(x, jnp.int32)
    magic = jnp.full((L,), 0x5F3759DF, jnp.int32)
    one = jnp.full((L,), 1, jnp.int32)
    y = lax.bitcast_convert_type(
        magic - lax.shift_right_arithmetic(i, one), jnp.float32)
    for _ in range(3):
        y = y * (1.5 - 0.5 * x * y * y)
    return y


def _lane_sum(x, lanes):
    for k in (8, 4, 2, 1):
        x = x + x.at[lanes ^ k].get(mode="promise_in_bounds")
    return x


def _shuf(x, lanes, k):
    return x.at[lanes ^ k].get(mode="promise_in_bounds")


def _merge(a, b, k, lanes):
    m = (lanes & k) != 0
    keep = jnp.where(m, b, a)
    give = jnp.where(m, a, b)
    return keep + _shuf(give, lanes, k)


def _reduce8(vs, lanes):
    for k in (1, 2, 4):
        vs = [_merge(vs[2 * i], vs[2 * i + 1], k, lanes)
              for i in range(len(vs) // 2)]
    z = vs[0]
    return z + _shuf(z, lanes, 8)


_mesh = plsc.VectorSubcoreMesh(
    core_axis_name="c", subcore_axis_name="s", num_cores=NC, num_subcores=NS
)


@functools.partial(
    pl.kernel,
    out_type=jax.ShapeDtypeStruct((N_LABELS * BATCH,), jnp.float32),
    mesh=_mesh,
    scratch_types=[
        pltpu.VMEM((RPW,), jnp.int32),
        pltpu.VMEM((3, N_DIM), jnp.float32),
        pltpu.VMEM((3, N_DIM), jnp.float32),
        pltpu.VMEM((2, K, N_DIM), jnp.float32),
        pltpu.VMEM((2, K, N_DIM), jnp.float32),
        pltpu.VMEM((2, K, N_DIM), jnp.float32),
        pltpu.VMEM(((RPW + 8) * N_LABELS,), jnp.float32),
        pltpu.SemaphoreType.DMA,
        pltpu.SemaphoreType.DMA,
    ],
)
def _fastvss_sc(qv_hbm, qc_hbm, pvs_hbm, qw_hbm, lab_hbm, idx_hbm, out_hbm,
                idx_v, qw_v, lab_v, pv_buf, qv_buf, qc_buf, out_v,
                sem0, sem1):
    wid = lax.axis_index("s") * NC + lax.axis_index("c")
    base = pl.multiple_of(wid * RPW, RPW)
    sems = (sem0, sem1)

    pltpu.sync_copy(idx_hbm.at[pl.ds(base, RPW)], idx_v)
    pltpu.sync_copy(qw_hbm, qw_v)
    pltpu.sync_copy(lab_hbm, lab_v)

    zero = jnp.zeros((L,), jnp.float32)
    lanes = lax.iota(jnp.int32, L)

    def pre_body(v, carry):
        sl = pl.ds(pl.multiple_of(v * L, L), L)
        for j in range(3):
            qw_v[j, sl] = qw_v[j, sl] * 2.0
        l0, l1, l2 = lab_v[0, sl], lab_v[1, sl], lab_v[2, sl]
        a0, a1, a2 = carry
        return (a0 + l0 * l0, a1 + l1 * l1, a2 + l2 * l2)

    la = lax.fori_loop(0, NCH, pre_body, (zero, zero, zero))
    inv_l = [_vrsqrt(_lane_sum(a, lanes)) for a in la]

    def lab_scale(v, c):
        sl = pl.ds(pl.multiple_of(v * L, L), L)
        for j in range(3):
            lab_v[j, sl] = lab_v[j, sl] * inv_l[j]
        return c

    lax.fori_loop(0, NCH, lab_scale, 0)

    def copies(slot, blk):
        off = base + blk * K
        return (
            pltpu.make_async_copy(
                pvs_hbm.at[idx_v.at[pl.ds(blk * K, K)]], pv_buf.at[slot], sems[slot]),
            pltpu.make_async_copy(
                qv_hbm.at[pl.ds(off, K)], qv_buf.at[slot], sems[slot]),
            pltpu.make_async_copy(
                qc_hbm.at[pl.ds(off, K)], qc_buf.at[slot], sems[slot]),
        )

    def issue(slot, blk):
        for c in copies(slot, blk):
            c.start()

    def wait(slot, blk):
        for c in copies(slot, blk):
            c.wait()

    def compute(slot, blk):
        pv_b, qv_b, qc_b = pv_buf.at[slot], qv_buf.at[slot], qc_buf.at[slot]

        def dim_body(v, carry):
            sl = pl.ds(pl.multiple_of(v * L, L), L)
            w0, w1, w2 = qw_v[0, sl], qw_v[1, sl], qw_v[2, sl]
            l0, l1, l2 = lab_v[0, sl], lab_v[1, sl], lab_v[2, sl]
            nxt = []
            for r in range(K):
                ss, d0, d1, d2 = carry[4 * r: 4 * r + 4]
                x = qv_b[r, sl] * w0 + qc_b[r, sl] * w1 + pv_b[r, sl] * w2
                t = 1.0 - 2.0 / (jnp.exp(x) + 1.0)
                nxt += [ss + t * t, d0 + t * l0, d1 + t * l1, d2 + t * l2]
            return tuple(nxt)

        accs = lax.fori_loop(0, NCH, dim_body, (zero,) * (4 * K))
        row0 = blk * K
        ssf = _reduce8([accs[4 * r + 0] for r in range(K)], lanes)
        inv_q = _vrsqrt(ssf)
        for j in range(N_LABELS):
            dj = _reduce8([accs[4 * r + 1 + j] for r in range(K)], lanes)
            out_v[pl.ds(j * (RPW + 8) + row0, L)] = dj * inv_q

    issue(0, 0)

    def outer(i2, c):
        b0 = i2 * 2
        issue(1, b0 + 1)
        wait(0, b0)
        compute(0, b0)

        @pl.when(b0 + 2 < NBLK)
        def _():
            issue(0, b0 + 2)

        wait(1, b0 + 1)
        compute(1, b0 + 1)
        return c

    lax.fori_loop(0, NBLK // 2, outer, 0)

    for j in range(N_LABELS):
        pltpu.sync_copy(
            out_v.at[pl.ds(j * (RPW + 8), RPW)],
            out_hbm.at[pl.ds(j * BATCH + base, RPW)])


def kernel(query_vec, qclass_vec, pvs, query_weight, label, product_idx):
    flat = _fastvss_sc(
        query_vec,
        qclass_vec,
        pvs,
        query_weight.astype(jnp.float32),
        label.astype(jnp.float32),
        product_idx.astype(jnp.int32),
    )
    return flat.reshape(N_LABELS, BATCH).T
